# Initial kernel scaffold; baseline (speedup 1.0000x reference)
import jax
import jax.numpy as jnp
from jax.experimental import pallas as pl


def kernel(edge_RWSE, batch, edge_index):
    raise NotImplementedError("write your pallas kernel here")


# speedup vs baseline: 3.0402x; 3.0402x over previous
import functools

import jax
import jax.numpy as jnp
from jax import lax
from jax.experimental import pallas as pl
from jax.experimental.pallas import tpu as pltpu
from jax.experimental.pallas import tpu_sc as plsc

EMB_DIM = 128


@functools.lru_cache(maxsize=None)
def _build(n, pw, E):
    info = plsc.get_sparse_core_info()
    NC, NS, L = info.num_cores, info.num_subcores, info.num_lanes
    NW = NC * NS
    BPW = E // NW
    CH = 128
    NCH = BPW // CH
    HR = BPW // 2
    ZC = EMB_DIM - pw

    mesh = plsc.VectorSubcoreMesh(core_axis_name="c", subcore_axis_name="s")

    def body(tab, src, dst, zsrc, out, src_v, dst_v, idx_v, rows_v, zer_v,
             sem, zsem):
        wid = lax.axis_index("s") * NC + lax.axis_index("c")
        base = wid * BPW
        zcp = pltpu.async_copy(zsrc, zer_v, zsem)
        pltpu.sync_copy(src.at[pl.ds(base, BPW)], src_v)
        pltpu.sync_copy(dst.at[pl.ds(base, BPW)], dst_v)
        for i in range(BPW // L):
            s16 = src_v[pl.ds(i * L, L)]
            d16 = dst_v[pl.ds(i * L, L)]
            idx_v[i // (CH // L), pl.ds((i % (CH // L)) * L, L)] = (
                s16 * n + lax.rem(d16, n))
        cps = [
            pltpu.async_copy(tab.at[idx_v.at[j]],
                             rows_v.at[pl.ds(j * CH, CH)], sem)
            for j in range(NCH)
        ]
        zcp.wait()
        pltpu.sync_copy(zer_v, out.at[pl.ds(base, HR), pl.ds(pw, ZC)])
        pltpu.sync_copy(zer_v, out.at[pl.ds(base + HR, HR), pl.ds(pw, ZC)])
        for cp in cps:
            cp.wait()
        pltpu.sync_copy(rows_v, out.at[pl.ds(base, BPW), pl.ds(0, pw)])

    return pl.kernel(
        body,
        mesh=mesh,
        out_type=jax.ShapeDtypeStruct((E, EMB_DIM), jnp.float32),
        scratch_types=[
            pltpu.VMEM((BPW,), jnp.int32),
            pltpu.VMEM((BPW,), jnp.int32),
            pltpu.VMEM((NCH, CH), jnp.int32),
            pltpu.VMEM((BPW, pw), jnp.float32),
            pltpu.VMEM((HR, ZC), jnp.float32),
            pltpu.SemaphoreType.DMA,
            pltpu.SemaphoreType.DMA,
        ],
        compiler_params=pltpu.CompilerParams(use_tc_tiling_on_sc=False),
    )


def kernel(edge_RWSE, batch, edge_index):
    total_nodes = batch.shape[0]
    n = edge_RWSE.shape[0] // total_nodes
    pe = edge_RWSE.shape[1]
    E = edge_index.shape[1]
    pw = ((pe + 7) // 8) * 8
    tab = jnp.pad(edge_RWSE, ((0, 0), (0, pw - pe)))
    ei = edge_index.astype(jnp.int32)
    zsrc = jnp.zeros((E // 64, EMB_DIM - pw), jnp.float32)
    return _build(n, pw, E)(tab, ei[0], ei[1], zsrc)
